# baseline (speedup 1.0000x reference)
import functools

import jax
import jax.numpy as jnp
from jax import lax
from jax.experimental import pallas as pl
from jax.experimental.pallas import tpu as pltpu
from jax.experimental.pallas import tpu_sc as plsc

N = 10000
D = 128
E = 320000

NC = 2
NSUB = 16
NW = NC * NSUB
ROWS_PER_SUB = 632
NPAD = NSUB * ROWS_PER_SUB
EDGES_PER_W = E // NW
CHUNK = 80
N_CHUNKS = 128
EDGES_PAD_W = N_CHUNKS * CHUNK
DUMMY = NPAD - 1

RBLK = 1000
GRID = N // RBLK

NS_ITERS = 22
HIGH = lax.Precision.HIGHEST


def _mm(a, b):
    return lax.dot_general(a, b, (((1,), (0,)), ((), ())),
                           precision=HIGH, preferred_element_type=jnp.float32)


def _outer(p, q):
    return lax.dot_general(p, q, (((0,), (0,)), ((), ())),
                           precision=HIGH, preferred_element_type=jnp.float32)


def _stats_update(step, h, g_ref, ca_ref, bv_ref, br_ref):
    a = jnp.abs(h)
    cs = jnp.sum(a, axis=0, keepdims=True)
    rs = jnp.sum(a, axis=1, keepdims=True)
    g = lax.dot_general(h, h, (((0,), (0,)), ((), ())),
                        precision=HIGH, preferred_element_type=jnp.float32)
    m = jnp.max(rs)
    ridx = lax.broadcasted_iota(jnp.int32, (RBLK, 1), 0)
    am = jnp.min(jnp.where(rs == m, ridx, RBLK))
    rmask = (ridx == am).astype(jnp.float32)
    row = jnp.sum(h * rmask, axis=0, keepdims=True)

    @pl.when(step == 0)
    def _():
        g_ref[...] = g
        ca_ref[...] = cs
        bv_ref[...] = jnp.full((1, 1), m, jnp.float32)
        br_ref[...] = row

    @pl.when(step != 0)
    def _():
        g_ref[...] += g
        ca_ref[...] += cs
        prev = bv_ref[0, 0]
        better = m > prev
        bv_ref[...] = jnp.full((1, 1), jnp.where(better, m, prev), jnp.float32)
        br_ref[...] = jnp.where(better, row, br_ref[...])


def _encoder_body(x_ref, wt_ref, b_ref, h_ref, g_ref, ca_ref, bv_ref, br_ref):
    i = pl.program_id(0)
    h = _mm(x_ref[...], wt_ref[...]) + b_ref[...]
    h_ref[...] = h
    _stats_update(i, h, g_ref, ca_ref, bv_ref, br_ref)


def _conv_body(relu, parts_ref, degp_ref, h_ref, wlt_ref, wrt_ref, b_ref,
               o_ref, g_ref, ca_ref, bv_ref, br_ref):
    i = pl.program_id(0)
    aggsum = parts_ref[0] + parts_ref[1]
    deg = degp_ref[0, :, 0:1] + degp_ref[1, :, 0:1]
    agg = aggsum / jnp.maximum(deg, 1.0)
    o = _mm(agg, wlt_ref[...]) + _mm(h_ref[...], wrt_ref[...]) + b_ref[...]
    if relu:
        o = jnp.maximum(o, 0.0)
    o_ref[...] = o
    _stats_update(i, o, g_ref, ca_ref, bv_ref, br_ref)


_STATS_OUT = [
    jax.ShapeDtypeStruct((N, D), jnp.float32),
    jax.ShapeDtypeStruct((D, D), jnp.float32),
    jax.ShapeDtypeStruct((1, D), jnp.float32),
    jax.ShapeDtypeStruct((1, 1), jnp.float32),
    jax.ShapeDtypeStruct((1, D), jnp.float32),
]
_STATS_SPECS = [
    pl.BlockSpec((RBLK, D), lambda i: (i, 0)),
    pl.BlockSpec((D, D), lambda i: (0, 0)),
    pl.BlockSpec((1, D), lambda i: (0, 0)),
    pl.BlockSpec((1, 1), lambda i: (0, 0)),
    pl.BlockSpec((1, D), lambda i: (0, 0)),
]


def _encoder(x, wt, b):
    return pl.pallas_call(
        _encoder_body,
        grid=(GRID,),
        in_specs=[
            pl.BlockSpec((RBLK, D), lambda i: (i, 0)),
            pl.BlockSpec((D, D), lambda i: (0, 0)),
            pl.BlockSpec((1, D), lambda i: (0, 0)),
        ],
        out_specs=_STATS_SPECS,
        out_shape=_STATS_OUT,
    )(x, wt, b)


def _conv(parts, degp, h, wlt, wrt, b, relu):
    return pl.pallas_call(
        functools.partial(_conv_body, relu),
        grid=(GRID,),
        in_specs=[
            pl.BlockSpec((NC, RBLK, D), lambda i: (0, i, 0)),
            pl.BlockSpec((NC, RBLK, D), lambda i: (0, i, 0)),
            pl.BlockSpec((RBLK, D), lambda i: (i, 0)),
            pl.BlockSpec((D, D), lambda i: (0, 0)),
            pl.BlockSpec((D, D), lambda i: (0, 0)),
            pl.BlockSpec((1, D), lambda i: (0, 0)),
        ],
        out_specs=_STATS_SPECS,
        out_shape=_STATS_OUT,
    )(parts, degp, h, wlt, wrt, b)


def _tracesqrt(A, eye):
    t = jnp.maximum(jnp.sum(A * eye), 1e-30)
    y0 = A / t

    def step(_, yz):
        y, z = yz
        m = 3.0 * eye - _mm(z, y)
        return 0.5 * _mm(y, m), 0.5 * _mm(m, z)

    y, _ = lax.fori_loop(0, NS_ITERS, step, (y0, eye))
    return jnp.sqrt(t) * jnp.sum(y * eye)


def _rankdiff_body(g_ref, r_ref, ca_ref, s_ref):
    G = g_ref[...]
    eye = (lax.broadcasted_iota(jnp.int32, (D, D), 0)
           == lax.broadcasted_iota(jnp.int32, (D, D), 1)).astype(jnp.float32)
    n0 = _tracesqrt(G, eye)

    ca = ca_ref[...]
    jcol = lax.broadcasted_iota(jnp.int32, (1, D), 1)
    mj = jnp.max(ca)
    jidx = jnp.min(jnp.where(ca == mj, jcol, D))
    ej = (jcol == jidx).astype(jnp.float32)

    gj = _mm(ej, G)
    Gjj = jnp.sum(gj * ej)
    r = r_ref[...]
    rj = jnp.sum(r * ej)
    sign = jnp.where(rj < 0.0, -1.0, 1.0)
    v = sign * r * lax.rsqrt(jnp.sum(r * r))
    a = gj / (n0 * jnp.sqrt(Gjj))

    dtd = G / (n0 * n0) - _outer(a, v) - _outer(v, a) + _outer(v, v)
    s_ref[...] = jnp.full((1, 1), _tracesqrt(dtd, eye), jnp.float32)


def _rankdiff(G, r, ca):
    return pl.pallas_call(
        _rankdiff_body,
        out_shape=jax.ShapeDtypeStruct((1, 1), jnp.float32),
    )(G, r, ca)


_SC_MESH = plsc.VectorSubcoreMesh(core_axis_name="c", subcore_axis_name="s")
_SC_OUT = [jax.ShapeDtypeStruct((NC, NPAD, D), jnp.float32)]


@functools.partial(pl.kernel, mesh=_SC_MESH, out_type=_SC_OUT,
                   scratch_types=[
                       pltpu.VMEM((CHUNK,), jnp.int32),
                       pltpu.VMEM((CHUNK,), jnp.int32),
                       pltpu.VMEM((CHUNK,), jnp.int32),
                       pltpu.VMEM((CHUNK,), jnp.int32),
                       pltpu.VMEM((CHUNK, D), jnp.float32),
                       pltpu.VMEM((CHUNK, D), jnp.float32),
                       pltpu.VMEM_SHARED((NPAD, D), jnp.float32),
                       pltpu.SemaphoreType.DMA,
                       pltpu.SemaphoreType.DMA,
                       pltpu.SemaphoreType.DMA,
                       pltpu.SemaphoreType.DMA,
                   ])
def _edgepass(h_hbm, src_hbm, dst_hbm, z_hbm, out_hbm,
              srcv0, srcv1, dstv0, dstv1, rows0, rows1, table,
              gsem0, gsem1, ssem0, ssem1):
    c = lax.axis_index("c")
    s = lax.axis_index("s")
    wid = s * NC + c
    row0 = pl.multiple_of(s * ROWS_PER_SUB, 8)

    pltpu.sync_copy(z_hbm.at[pl.ds(row0, ROWS_PER_SUB)],
                    table.at[pl.ds(row0, ROWS_PER_SUB)])
    plsc.subcore_barrier()

    srcv = [srcv0, srcv1]
    dstv = [dstv0, dstv1]
    rows = [rows0, rows1]
    gsem = [gsem0, gsem1]
    ssem = [ssem0, ssem1]
    base = wid * EDGES_PAD_W

    def outer(m, carry):
        gh = []
        for b in range(2):
            off = pl.multiple_of(base + (m * 2 + b) * CHUNK, 8)
            pltpu.sync_copy(src_hbm.at[pl.ds(off, CHUNK)], srcv[b])
            pltpu.sync_copy(dst_hbm.at[pl.ds(off, CHUNK)], dstv[b])
            gh.append(pltpu.async_copy(h_hbm.at[srcv[b]], rows[b], gsem[b]))
        sh = []
        for b in range(2):
            gh[b].wait()
            sh.append(pltpu.async_copy(rows[b], table.at[dstv[b]], ssem[b],
                                       add=True))
        for hnd in sh:
            hnd.wait()
        return carry

    lax.fori_loop(0, N_CHUNKS // 2, outer, 0)
    plsc.subcore_barrier()

    pltpu.sync_copy(table.at[pl.ds(row0, ROWS_PER_SUB)],
                    out_hbm.at[c, pl.ds(row0, ROWS_PER_SUB)])


@functools.partial(pl.kernel, mesh=_SC_MESH, out_type=_SC_OUT,
                   scratch_types=[
                       pltpu.VMEM((CHUNK,), jnp.int32),
                       pltpu.VMEM((CHUNK,), jnp.int32),
                       pltpu.VMEM((CHUNK, D), jnp.float32),
                       pltpu.VMEM_SHARED((NPAD, D), jnp.float32),
                       pltpu.SemaphoreType.DMA,
                       pltpu.SemaphoreType.DMA,
                   ])
def _degpass(dst_hbm, z_hbm, ones_hbm, out_hbm, dstv0, dstv1, onesv, table,
             ssem0, ssem1):
    c = lax.axis_index("c")
    s = lax.axis_index("s")
    wid = s * NC + c
    row0 = pl.multiple_of(s * ROWS_PER_SUB, 8)

    pltpu.sync_copy(z_hbm.at[pl.ds(row0, ROWS_PER_SUB)],
                    table.at[pl.ds(row0, ROWS_PER_SUB)])
    pltpu.sync_copy(ones_hbm, onesv)
    plsc.subcore_barrier()

    dstv = [dstv0, dstv1]
    ssem = [ssem0, ssem1]
    base = wid * EDGES_PAD_W

    def outer(m, carry):
        sh = []
        for b in range(2):
            off = pl.multiple_of(base + (m * 2 + b) * CHUNK, 8)
            pltpu.sync_copy(dst_hbm.at[pl.ds(off, CHUNK)], dstv[b])
            sh.append(pltpu.async_copy(onesv, table.at[dstv[b]],
                                       ssem[b], add=True))
        for hnd in sh:
            hnd.wait()
        return carry

    lax.fori_loop(0, N_CHUNKS // 2, outer, 0)
    plsc.subcore_barrier()

    pltpu.sync_copy(table.at[pl.ds(row0, ROWS_PER_SUB)],
                    out_hbm.at[c, pl.ds(row0, ROWS_PER_SUB)])


def kernel(x, edge_index, W_enc, b_enc, Wl0, Wr0, b0, Wl1, Wr1, b1):
    pad_w = EDGES_PAD_W - EDGES_PER_W
    src_p = jnp.concatenate(
        [edge_index[0].reshape(NW, EDGES_PER_W),
         jnp.zeros((NW, pad_w), jnp.int32)], axis=1,
    ).reshape(NW * EDGES_PAD_W)
    dst_p = jnp.concatenate(
        [edge_index[1].reshape(NW, EDGES_PER_W),
         jnp.full((NW, pad_w), DUMMY, jnp.int32)], axis=1,
    ).reshape(NW * EDGES_PAD_W)
    zeros_t = jnp.zeros((NPAD, D), jnp.float32)
    ones_c = jnp.ones((CHUNK, D), jnp.float32)

    h0, G0, ca0, _, br0 = _encoder(x, W_enc.T, b_enc.reshape(1, D))
    s0 = _rankdiff(G0, br0, ca0)

    (degp,) = _degpass(dst_p, zeros_t, ones_c)
    (parts0,) = _edgepass(h0, src_p, dst_p, zeros_t)
    h1, G1, ca1, _, br1 = _conv(parts0, degp, h0, Wl0.T, Wr0.T,
                                b0.reshape(1, D), relu=True)
    s1 = _rankdiff(G1, br1, ca1)

    (parts1,) = _edgepass(h1, src_p, dst_p, zeros_t)
    h2, G2, ca2, _, br2 = _conv(parts1, degp, h1, Wl1.T, Wr1.T,
                                b1.reshape(1, D), relu=False)
    s2 = _rankdiff(G2, br2, ca2)

    return h2, jnp.stack([s0[0, 0], s1[0, 0], s2[0, 0]])


# speedup vs baseline: 1.4092x; 1.4092x over previous
import functools

import jax
import jax.numpy as jnp
from jax import lax
from jax.experimental import pallas as pl
from jax.experimental.pallas import tpu as pltpu
from jax.experimental.pallas import tpu_sc as plsc

N = 10000
D = 128
E = 320000

NC = 2
NSUB = 16
NW = NC * NSUB
ROWS_PER_SUB = 632
NPAD = NSUB * ROWS_PER_SUB
EDGES_PER_W = E // NW
CHUNK = 80
N_CHUNKS = EDGES_PER_W // CHUNK

RBLK = 1000
GRID = N // RBLK

NS_ITERS = 22
HIGH = lax.Precision.HIGHEST


def _mm(a, b):
    return lax.dot_general(a, b, (((1,), (0,)), ((), ())),
                           precision=HIGH, preferred_element_type=jnp.float32)


def _outer(p, q):
    return lax.dot_general(p, q, (((0,), (0,)), ((), ())),
                           precision=HIGH, preferred_element_type=jnp.float32)


def _stats_update(step, h, g_ref, ca_ref, bv_ref, br_ref):
    a = jnp.abs(h)
    cs = jnp.sum(a, axis=0, keepdims=True)
    rs = jnp.sum(a, axis=1, keepdims=True)
    g = lax.dot_general(h, h, (((0,), (0,)), ((), ())),
                        precision=HIGH, preferred_element_type=jnp.float32)
    m = jnp.max(rs)
    ridx = lax.broadcasted_iota(jnp.int32, (RBLK, 1), 0)
    am = jnp.min(jnp.where(rs == m, ridx, RBLK))
    rmask = (ridx == am).astype(jnp.float32)
    row = jnp.sum(h * rmask, axis=0, keepdims=True)

    @pl.when(step == 0)
    def _():
        g_ref[...] = g
        ca_ref[...] = cs
        bv_ref[...] = jnp.full((1, 1), m, jnp.float32)
        br_ref[...] = row

    @pl.when(step != 0)
    def _():
        g_ref[...] += g
        ca_ref[...] += cs
        prev = bv_ref[0, 0]
        better = m > prev
        bv_ref[...] = jnp.full((1, 1), jnp.where(better, m, prev), jnp.float32)
        br_ref[...] = jnp.where(better, row, br_ref[...])


def _encoder_body(x_ref, wt_ref, b_ref, h_ref, g_ref, ca_ref, bv_ref, br_ref):
    i = pl.program_id(0)
    h = _mm(x_ref[...], wt_ref[...]) + b_ref[...]
    h_ref[...] = h
    _stats_update(i, h, g_ref, ca_ref, bv_ref, br_ref)


def _conv_body(relu, parts_ref, degp_ref, h_ref, wlt_ref, wrt_ref, b_ref,
               o_ref, g_ref, ca_ref, bv_ref, br_ref):
    i = pl.program_id(0)
    aggsum = parts_ref[0] + parts_ref[1]
    deg = degp_ref[0, :, 0:1] + degp_ref[1, :, 0:1]
    agg = aggsum / jnp.maximum(deg, 1.0)
    o = _mm(agg, wlt_ref[...]) + _mm(h_ref[...], wrt_ref[...]) + b_ref[...]
    if relu:
        o = jnp.maximum(o, 0.0)
    o_ref[...] = o
    _stats_update(i, o, g_ref, ca_ref, bv_ref, br_ref)


_STATS_OUT = [
    jax.ShapeDtypeStruct((N, D), jnp.float32),
    jax.ShapeDtypeStruct((D, D), jnp.float32),
    jax.ShapeDtypeStruct((1, D), jnp.float32),
    jax.ShapeDtypeStruct((1, 1), jnp.float32),
    jax.ShapeDtypeStruct((1, D), jnp.float32),
]
_STATS_SPECS = [
    pl.BlockSpec((RBLK, D), lambda i: (i, 0)),
    pl.BlockSpec((D, D), lambda i: (0, 0)),
    pl.BlockSpec((1, D), lambda i: (0, 0)),
    pl.BlockSpec((1, 1), lambda i: (0, 0)),
    pl.BlockSpec((1, D), lambda i: (0, 0)),
]


def _encoder(x, wt, b):
    return pl.pallas_call(
        _encoder_body,
        grid=(GRID,),
        in_specs=[
            pl.BlockSpec((RBLK, D), lambda i: (i, 0)),
            pl.BlockSpec((D, D), lambda i: (0, 0)),
            pl.BlockSpec((1, D), lambda i: (0, 0)),
        ],
        out_specs=_STATS_SPECS,
        out_shape=_STATS_OUT,
    )(x, wt, b)


def _conv(parts, degp, h, wlt, wrt, b, relu):
    return pl.pallas_call(
        functools.partial(_conv_body, relu),
        grid=(GRID,),
        in_specs=[
            pl.BlockSpec((NC, RBLK, D), lambda i: (0, i, 0)),
            pl.BlockSpec((NC, RBLK, D), lambda i: (0, i, 0)),
            pl.BlockSpec((RBLK, D), lambda i: (i, 0)),
            pl.BlockSpec((D, D), lambda i: (0, 0)),
            pl.BlockSpec((D, D), lambda i: (0, 0)),
            pl.BlockSpec((1, D), lambda i: (0, 0)),
        ],
        out_specs=_STATS_SPECS,
        out_shape=_STATS_OUT,
    )(parts, degp, h, wlt, wrt, b)


def _tracesqrt(A, eye):
    t = jnp.maximum(jnp.sum(A * eye), 1e-30)
    y0 = A / t

    def step(_, yz):
        y, z = yz
        m = 3.0 * eye - _mm(z, y)
        return 0.5 * _mm(y, m), 0.5 * _mm(m, z)

    y, _ = lax.fori_loop(0, NS_ITERS, step, (y0, eye))
    return jnp.sqrt(t) * jnp.sum(y * eye)


def _rankdiff_body(g_ref, r_ref, ca_ref, s_ref):
    G = g_ref[...]
    eye = (lax.broadcasted_iota(jnp.int32, (D, D), 0)
           == lax.broadcasted_iota(jnp.int32, (D, D), 1)).astype(jnp.float32)
    n0 = _tracesqrt(G, eye)

    ca = ca_ref[...]
    jcol = lax.broadcasted_iota(jnp.int32, (1, D), 1)
    mj = jnp.max(ca)
    jidx = jnp.min(jnp.where(ca == mj, jcol, D))
    ej = (jcol == jidx).astype(jnp.float32)

    gj = _mm(ej, G)
    Gjj = jnp.sum(gj * ej)
    r = r_ref[...]
    rj = jnp.sum(r * ej)
    sign = jnp.where(rj < 0.0, -1.0, 1.0)
    v = sign * r * lax.rsqrt(jnp.sum(r * r))
    a = gj / (n0 * jnp.sqrt(Gjj))

    dtd = G / (n0 * n0) - _outer(a, v) - _outer(v, a) + _outer(v, v)
    s_ref[...] = jnp.full((1, 1), _tracesqrt(dtd, eye), jnp.float32)


def _rankdiff(G, r, ca):
    return pl.pallas_call(
        _rankdiff_body,
        out_shape=jax.ShapeDtypeStruct((1, 1), jnp.float32),
    )(G, r, ca)


_SC_MESH = plsc.VectorSubcoreMesh(core_axis_name="c", subcore_axis_name="s")
_SC_OUT = [jax.ShapeDtypeStruct((NC, NPAD, D), jnp.float32)]


@functools.partial(pl.kernel, mesh=_SC_MESH, out_type=_SC_OUT,
                   scratch_types=[
                       pltpu.VMEM((CHUNK,), jnp.int32),
                       pltpu.VMEM((CHUNK,), jnp.int32),
                       pltpu.VMEM((CHUNK, D), jnp.float32),
                       pltpu.VMEM_SHARED((NPAD, D), jnp.float32),
                       pltpu.SemaphoreType.DMA,
                   ])
def _edgepass(h_hbm, src_hbm, dst_hbm, z_hbm, out_hbm,
              srcv, dstv, rows, table, sem):
    c = lax.axis_index("c")
    s = lax.axis_index("s")
    wid = s * NC + c
    row0 = pl.multiple_of(s * ROWS_PER_SUB, 8)

    pltpu.sync_copy(z_hbm.at[pl.ds(row0, ROWS_PER_SUB)],
                    table.at[pl.ds(row0, ROWS_PER_SUB)])
    plsc.subcore_barrier()

    base = wid * EDGES_PER_W

    def body(j, carry):
        off = pl.multiple_of(base + j * CHUNK, 8)
        pltpu.sync_copy(src_hbm.at[pl.ds(off, CHUNK)], srcv)
        pltpu.sync_copy(dst_hbm.at[pl.ds(off, CHUNK)], dstv)
        pltpu.async_copy(h_hbm.at[srcv], rows, sem).wait()
        pltpu.sync_copy(rows, table.at[dstv], add=True)
        return carry

    lax.fori_loop(0, N_CHUNKS, body, 0)
    plsc.subcore_barrier()

    pltpu.sync_copy(table.at[pl.ds(row0, ROWS_PER_SUB)],
                    out_hbm.at[c, pl.ds(row0, ROWS_PER_SUB)])


@functools.partial(pl.kernel, mesh=_SC_MESH, out_type=_SC_OUT,
                   scratch_types=[
                       pltpu.VMEM((CHUNK,), jnp.int32),
                       pltpu.VMEM((CHUNK, D), jnp.float32),
                       pltpu.VMEM_SHARED((NPAD, D), jnp.float32),
                   ])
def _degpass(dst_hbm, z_hbm, ones_hbm, out_hbm, dstv, onesv, table):
    c = lax.axis_index("c")
    s = lax.axis_index("s")
    wid = s * NC + c
    row0 = pl.multiple_of(s * ROWS_PER_SUB, 8)

    pltpu.sync_copy(z_hbm.at[pl.ds(row0, ROWS_PER_SUB)],
                    table.at[pl.ds(row0, ROWS_PER_SUB)])
    pltpu.sync_copy(ones_hbm, onesv)
    plsc.subcore_barrier()

    base = wid * EDGES_PER_W

    def body(j, carry):
        off = pl.multiple_of(base + j * CHUNK, 8)
        pltpu.sync_copy(dst_hbm.at[pl.ds(off, CHUNK)], dstv)
        pltpu.sync_copy(onesv, table.at[dstv], add=True)
        return carry

    lax.fori_loop(0, N_CHUNKS, body, 0)
    plsc.subcore_barrier()

    pltpu.sync_copy(table.at[pl.ds(row0, ROWS_PER_SUB)],
                    out_hbm.at[c, pl.ds(row0, ROWS_PER_SUB)])


def kernel(x, edge_index, W_enc, b_enc, Wl0, Wr0, b0, Wl1, Wr1, b1):
    src_p = edge_index[0]
    dst_p = edge_index[1]
    zeros_t = jnp.zeros((NPAD, D), jnp.float32)
    ones_c = jnp.ones((CHUNK, D), jnp.float32)

    h0, G0, ca0, _, br0 = _encoder(x, W_enc.T, b_enc.reshape(1, D))
    s0 = _rankdiff(G0, br0, ca0)

    (degp,) = _degpass(dst_p, zeros_t, ones_c)
    (parts0,) = _edgepass(h0, src_p, dst_p, zeros_t)
    h1, G1, ca1, _, br1 = _conv(parts0, degp, h0, Wl0.T, Wr0.T,
                                b0.reshape(1, D), relu=True)
    s1 = _rankdiff(G1, br1, ca1)

    (parts1,) = _edgepass(h1, src_p, dst_p, zeros_t)
    h2, G2, ca2, _, br2 = _conv(parts1, degp, h1, Wl1.T, Wr1.T,
                                b1.reshape(1, D), relu=False)
    s2 = _rankdiff(G2, br2, ca2)

    return h2, jnp.stack([s0[0, 0], s1[0, 0], s2[0, 0]])
